# Initial kernel scaffold; baseline (speedup 1.0000x reference)
import jax
import jax.numpy as jnp
from jax.experimental import pallas as pl


def kernel(batch, table):
    raise NotImplementedError("write your pallas kernel here")


# speedup vs baseline: 1.3647x; 1.3647x over previous
import functools

import jax
import jax.numpy as jnp
from jax import lax
from jax.experimental import pallas as pl
from jax.experimental.pallas import tpu as pltpu
from jax.experimental.pallas import tpu_sc as plsc

_CHUNK = 128
_LANES = 128


@functools.partial(jax.jit, static_argnames=("nc", "ns", "d"))
def _lookup(idx, table_pad, d, nc, ns):
    nw = nc * ns
    _, n_ch, _ = idx.shape
    _, dp = table_pad.shape
    per_w = n_ch * _CHUNK
    n = nw * per_w
    mesh = plsc.VectorSubcoreMesh(core_axis_name="c", subcore_axis_name="s")

    @functools.partial(
        pl.kernel,
        mesh=mesh,
        out_type=jax.ShapeDtypeStruct((n, dp), jnp.float32),
        scratch_types=[
            pltpu.VMEM((n_ch, _CHUNK), jnp.int32),
            pltpu.VMEM((2, _CHUNK, dp), jnp.float32),
            pltpu.SemaphoreType.DMA,
            pltpu.SemaphoreType.DMA,
        ],
    )
    def lookup(idx_hbm, table_hbm, out_hbm, idx_v, rows_v, sem0, sem1):
        wid = lax.axis_index("s") * nc + lax.axis_index("c")
        base = wid * per_w
        pltpu.sync_copy(idx_hbm.at[wid], idx_v)
        sems = (sem0, sem1)
        for b in range(2):
            pltpu.async_copy(table_hbm.at[idx_v.at[b]], rows_v.at[b], sems[b])

        def step(i, carry):
            g = i * 2
            for b in range(2):
                ch = g + b
                pltpu.make_async_copy(
                    table_hbm.at[idx_v.at[ch]], rows_v.at[b], sems[b]
                ).wait()
                pltpu.sync_copy(
                    rows_v.at[b],
                    out_hbm.at[pl.ds(base + ch * _CHUNK, _CHUNK)],
                )

                @pl.when(ch + 2 < n_ch)
                def _():
                    pltpu.async_copy(
                        table_hbm.at[idx_v.at[ch + 2]], rows_v.at[b], sems[b]
                    )

            return carry

        lax.fori_loop(0, n_ch // 2, step, 0)

    return lookup(idx, table_pad)


def kernel(batch, table):
    b, s = batch.shape
    _, d = table.shape
    n = b * s
    dp = ((d + _LANES - 1) // _LANES) * _LANES
    info = plsc.get_sparse_core_info()
    nc, ns = info.num_cores, info.num_subcores
    nw = nc * ns
    n_ch = n // (nw * _CHUNK)
    idx = batch.reshape(nw, n_ch, _CHUNK)
    table_pad = jnp.pad(table, ((0, 0), (0, dp - d)))
    out = _lookup(idx, table_pad, d, nc, ns)
    return out[:, :d].reshape(b, s, d)


# speedup vs baseline: 1.6402x; 1.2019x over previous
import functools

import jax
import jax.numpy as jnp
from jax import lax
from jax.experimental import pallas as pl
from jax.experimental.pallas import tpu as pltpu
from jax.experimental.pallas import tpu_sc as plsc

_CHUNK = 64
_ALIGNED = 256


@functools.partial(jax.jit, static_argnames=("nc", "ns", "d"))
def _lookup(idx, table, table_tail, d, nc, ns):
    nw = nc * ns
    _, n_ch, _ = idx.shape
    per_w = n_ch * _CHUNK
    n = nw * per_w
    tail = d - _ALIGNED
    mesh = plsc.VectorSubcoreMesh(core_axis_name="c", subcore_axis_name="s")

    @functools.partial(
        pl.kernel,
        mesh=mesh,
        out_type=jax.ShapeDtypeStruct((n, d), jnp.float32),
        compiler_params=pltpu.CompilerParams(needs_layout_passes=False),
        scratch_types=[
            pltpu.VMEM((n_ch, _CHUNK), jnp.int32),
            pltpu.VMEM((2, _CHUNK, d), jnp.float32),
            pltpu.VMEM((2, _CHUNK, 128), jnp.float32),
            pltpu.SemaphoreType.DMA,
            pltpu.SemaphoreType.DMA,
            pltpu.SemaphoreType.DMA,
            pltpu.SemaphoreType.DMA,
        ],
    )
    def lookup(
        idx_hbm, table_hbm, tail_hbm, out_hbm,
        idx_v, rows_v, tail_v, semA0, semA1, semB0, semB1,
    ):
        wid = lax.axis_index("s") * nc + lax.axis_index("c")
        base = wid * per_w
        pltpu.sync_copy(idx_hbm.at[wid], idx_v)
        semsA = (semA0, semA1)
        semsB = (semB0, semB1)
        table_main = table_hbm.at[:, pl.ds(0, _ALIGNED)]
        lanes = lax.iota(jnp.int32, 16)
        tail_rem = tail - 32
        rem_mask = lanes < tail_rem
        rem_cols = _ALIGNED + 32 + lanes

        def start(ch, b):
            pltpu.async_copy(
                table_main.at[idx_v.at[ch]],
                rows_v.at[b, :, pl.ds(0, _ALIGNED)],
                semsA[b],
            )
            pltpu.async_copy(tail_hbm.at[idx_v.at[ch]], tail_v.at[b], semsB[b])

        def wait(ch, b):
            pltpu.make_async_copy(
                table_main.at[idx_v.at[ch]],
                rows_v.at[b, :, pl.ds(0, _ALIGNED)],
                semsA[b],
            ).wait()
            pltpu.make_async_copy(
                tail_hbm.at[idx_v.at[ch]], tail_v.at[b], semsB[b]
            ).wait()

        for b in range(2):
            start(b, b)

        def step(i, carry):
            g = i * 2
            for b in range(2):
                ch = g + b
                wait(ch, b)
                b_ix = jnp.full((16,), b, jnp.int32)

                def repack(r, c2):
                    rows_v[b, r, pl.ds(_ALIGNED, 16)] = tail_v[b, r, pl.ds(0, 16)]
                    rows_v[b, r, pl.ds(_ALIGNED + 16, 16)] = tail_v[
                        b, r, pl.ds(16, 16)
                    ]
                    plsc.store_scatter(
                        rows_v,
                        [b_ix, jnp.full((16,), r, jnp.int32), rem_cols],
                        tail_v[b, r, pl.ds(32, 16)],
                        mask=rem_mask,
                    )
                    return c2

                lax.fori_loop(0, _CHUNK, repack, 0)
                pltpu.sync_copy(
                    rows_v.at[b],
                    out_hbm.at[pl.ds(base + ch * _CHUNK, _CHUNK)],
                )

                @pl.when(ch + 2 < n_ch)
                def _():
                    start(ch + 2, b)

            return carry

        lax.fori_loop(0, n_ch // 2, step, 0)

    return lookup(idx, table, table_tail)


def kernel(batch, table):
    b, s = batch.shape
    _, d = table.shape
    n = b * s
    info = plsc.get_sparse_core_info()
    nc, ns = info.num_cores, info.num_subcores
    nw = nc * ns
    n_ch = n // (nw * _CHUNK)
    idx = batch.reshape(nw, n_ch, _CHUNK)
    tail = d - _ALIGNED
    table_tail = jnp.pad(table[:, _ALIGNED:], ((0, 0), (0, 128 - tail)))
    out = _lookup(idx, table, table_tail, d, nc, ns)
    return out.reshape(b, s, d)
